# baseline (speedup 1.0000x reference)
import functools

import jax
import jax.numpy as jnp
from jax import lax
from jax.experimental import pallas as pl
from jax.experimental.pallas import tpu as pltpu
from jax.experimental.pallas import tpu_sc as plsc

BATCH = 16384
FIELDS = 26
D = 64
B = BATCH * FIELDS
NW = 32
GROUP = 512
ROWS_PER_W = B // NW
NGROUP = ROWS_PER_W // GROUP
NBUF = 2

_COEFS = (
    -9.8719611294202e-07,
    1.8192777221918577e-05,
    -0.00020655130351230762,
    0.002080658900148311,
    -0.020832713479810427,
    0.24999997673756713,
)


@functools.partial(
    pl.kernel,
    out_type=jax.ShapeDtypeStruct((B, D), jnp.float32),
    mesh=plsc.VectorSubcoreMesh(core_axis_name="c", subcore_axis_name="s"),
    scratch_types=[
        pltpu.VMEM((ROWS_PER_W,), jnp.int32),
        [pltpu.VMEM((GROUP, D), jnp.float32) for _ in range(NBUF)],
        [pltpu.SemaphoreType.DMA for _ in range(NBUF)],
        [pltpu.SemaphoreType.DMA for _ in range(NBUF)],
    ],
    compiler_params=pltpu.CompilerParams(use_tc_tiling_on_sc=False),
)
def _emb_swish(idx_hbm, table_hbm, out_hbm, idx_v, bufs, gsem, ssem):
    wid = lax.axis_index("s") * 2 + lax.axis_index("c")
    pltpu.sync_copy(idx_hbm.at[pl.ds(wid * ROWS_PER_W, ROWS_PER_W)], idx_v)

    def gather_start(g, b):
        def vec(v, c):
            idxv = idx_v[pl.ds(g * GROUP + v * 16, 16)]
            for k in range(16):
                s = idxv[k]
                pltpu.make_async_copy(
                    table_hbm.at[pl.ds(s, 1)],
                    bufs[b].at[pl.ds(v * 16 + k, 1)],
                    gsem[b],
                ).start()
            return c

        lax.fori_loop(0, GROUP // 16, vec, 0)

    def gather_wait(g, b):
        pltpu.make_async_copy(
            table_hbm.at[pl.ds(0, GROUP)], bufs[b], gsem[b]
        ).wait()

    def store(g, b):
        return pltpu.make_async_copy(
            bufs[b],
            out_hbm.at[pl.ds((wid * NGROUP + g) * GROUP, GROUP)],
            ssem[b],
        )

    gather_start(0, 0)

    def outer(i, carry):
        for j in range(NBUF):
            g = i * NBUF + j
            b2 = 1 - j

            @pl.when((g >= 1) & (g + 1 < NGROUP))
            def _():
                store(g - 1, b2).wait()

            @pl.when(g + 1 < NGROUP)
            def _():
                gather_start(g + 1, b2)

            gather_wait(g, j)

            def row_body(r, carry2, _j=j):
                for t in range(D // 16):
                    v = bufs[_j][r, pl.ds(t * 16, 16)]
                    u = v * v
                    q = _COEFS[0]
                    for coef in _COEFS[1:]:
                        q = q * u + coef
                    bufs[_j][r, pl.ds(t * 16, 16)] = 0.5 * v + u * q
                return carry2

            lax.fori_loop(0, GROUP, row_body, 0)
            store(g, j).start()

        return carry

    lax.fori_loop(0, NGROUP // NBUF, outer, 0)
    store(NGROUP - 2, 0).wait()
    store(NGROUP - 1, 1).wait()


def kernel(x, emb_weight):
    idx = x.astype(jnp.int32).reshape(B)
    out = _emb_swish(idx, emb_weight)
    return out.reshape(BATCH, FIELDS, D)
